# Initial kernel scaffold; baseline (speedup 1.0000x reference)
import jax
import jax.numpy as jnp
from jax.experimental import pallas as pl


def kernel(input, confound_cat, outcome_y, W_in, Wc1, Wc2, Wf):
    raise NotImplementedError("write your pallas kernel here")


# speedup vs baseline: 106.9658x; 106.9658x over previous
import functools

import jax
import jax.numpy as jnp
from jax import lax
from jax.experimental import pallas as pl
from jax.experimental.pallas import tpu as pltpu
from jax.experimental.pallas import tpu_sc as plsc

VOCAB = 1000
HIDDEN = 128
B = 16384
L = 200

_NC = 2
_NS = 16
_NW = _NC * _NS
_R = B // _NW
_VPAD = 1024


def _prep_body(wf_ref, win_ref, wc1_ref, wc2_ref, u_ref, vw_ref):
    wf_text = wf_ref[0:1, 0:HIDDEN]
    u = jnp.dot(wf_text, win_ref[...],
                preferred_element_type=jnp.float32,
                precision=lax.Precision.HIGHEST)
    col = lax.broadcasted_iota(jnp.int32, (1, _VPAD), 1)
    u_ref[...] = jnp.where(col == 1, 0.0, u)
    v = jnp.dot(wc2_ref[...], wc1_ref[...],
                preferred_element_type=jnp.float32,
                precision=lax.Precision.HIGHEST)
    colv = lax.broadcasted_iota(jnp.int32, (1, HIDDEN), 1)
    vm = jnp.where(colv == 1, 0.0, v)
    w2 = vm * wf_ref[0:1, HIDDEN:HIDDEN + 1]
    vw_ref[...] = jnp.concatenate([vm, w2], axis=0)


_prep = pl.pallas_call(
    _prep_body,
    out_shape=(
        jax.ShapeDtypeStruct((1, _VPAD), jnp.float32),
        jax.ShapeDtypeStruct((2, HIDDEN), jnp.float32),
    ),
)


def _sc_main_body(ids_hbm, cat_hbm, y_hbm, u_hbm, vw_hbm,
             conf_hbm, fin_hbm, loss_hbm,
                  ids_v, u_v, vw_v, cat_v, y_v, t_v, conf_v, fin_v, loss_v,
                  sem):
    wid = lax.axis_index("s") * _NC + lax.axis_index("c")
    base = wid * _R

    big = pltpu.async_copy(ids_hbm.at[pl.ds(base * L, _R * L)],
                           ids_v.at[pl.ds(0, _R * L)], sem)
    pltpu.sync_copy(u_hbm, u_v)
    pltpu.sync_copy(vw_hbm, vw_v)
    pltpu.sync_copy(cat_hbm.at[pl.ds(base, _R)], cat_v)
    pltpu.sync_copy(y_hbm.at[pl.ds(base, _R)], y_v)
    big.wait()
    ids_v[pl.ds(_R * L, 16)] = jnp.zeros((16,), jnp.int32)

    lane = lax.iota(jnp.int32, 16)
    tail_mask = lane < 8
    last_mask = lane == 15

    def row_body(r, carry):
        off = r * L
        acc0 = plsc.load_gather(u_v, [ids_v[pl.ds(off, 16)]])
        acc1 = plsc.load_gather(u_v, [ids_v[pl.ds(off + 16, 16)]])
        for jv in range(2, 12, 2):
            acc0 = acc0 + plsc.load_gather(u_v, [ids_v[pl.ds(off + jv * 16, 16)]])
            acc1 = acc1 + plsc.load_gather(u_v, [ids_v[pl.ds(off + jv * 16 + 16, 16)]])
        tail = plsc.load_gather(u_v, [ids_v[pl.ds(off + 192, 16)]])
        acc = acc0 + acc1 + jnp.where(tail_mask, tail, 0.0)
        csum = plsc.cumsum(acc)
        plsc.store_scatter(t_v, [jnp.full((16,), r, jnp.int32)], csum,
                           mask=last_mask)
        return carry

    lax.fori_loop(0, _R, row_body, 0, unroll=False)

    zero16 = jnp.zeros((16,), jnp.float32)

    def ep_body(g, carry):
        lc, lf = carry
        t16 = t_v[pl.ds(g * 16, 16)]
        c16 = cat_v[pl.ds(g * 16, 16)]
        y16 = y_v[pl.ds(g * 16, 16)]
        cpv = plsc.load_gather(vw_v, [c16])
        w2v = plsc.load_gather(vw_v, [c16 + HIDDEN])
        fpv = t16 + w2v
        conf_v[pl.ds(g * 16, 16)] = cpv
        fin_v[pl.ds(g * 16, 16)] = fpv
        dc = cpv - y16
        df = fpv - y16
        return (lc + dc * dc, lf + df * df)

    lc, lf = lax.fori_loop(0, _R // 16, ep_body, (zero16, zero16),
                           unroll=False)
    loss_v[pl.ds(0, 16)] = lc
    loss_v[pl.ds(16, 16)] = lf

    pltpu.sync_copy(conf_v, conf_hbm.at[pl.ds(base, _R)])
    pltpu.sync_copy(fin_v, fin_hbm.at[pl.ds(base, _R)])
    pltpu.sync_copy(loss_v, loss_hbm.at[pl.ds(wid * 32, 32)])


@functools.cache
def _get_sc_main():
    mesh = plsc.VectorSubcoreMesh(core_axis_name="c", subcore_axis_name="s",
                                  num_cores=_NC, num_subcores=_NS)
    return pl.kernel(
        _sc_main_body,
        mesh=mesh,
        compiler_params=pltpu.CompilerParams(needs_layout_passes=False),
        out_type=(
            jax.ShapeDtypeStruct((B,), jnp.float32),
            jax.ShapeDtypeStruct((B,), jnp.float32),
            jax.ShapeDtypeStruct((_NW * 32,), jnp.float32),
        ),
        scratch_types=[
            pltpu.VMEM((_R * L + 16,), jnp.int32),
            pltpu.VMEM((_VPAD,), jnp.float32),
            pltpu.VMEM((2 * HIDDEN,), jnp.float32),
            pltpu.VMEM((_R,), jnp.int32),
            pltpu.VMEM((_R,), jnp.float32),
            pltpu.VMEM((_R,), jnp.float32),
            pltpu.VMEM((_R,), jnp.float32),
            pltpu.VMEM((_R,), jnp.float32),
            pltpu.VMEM((32,), jnp.float32),
            pltpu.SemaphoreType.DMA,
        ],
    )


def kernel(input, confound_cat, outcome_y, W_in, Wc1, Wc2, Wf):
    ids = input.astype(jnp.int32).reshape(B * L)
    cat = confound_cat.astype(jnp.int32)
    y = outcome_y.astype(jnp.float32)

    win_pad = jnp.zeros((HIDDEN, _VPAD), jnp.float32).at[:, :VOCAB].set(W_in)
    wc1_pad = jnp.zeros((HIDDEN, HIDDEN), jnp.float32).at[:, :10].set(Wc1)

    u2d, vw2d = _prep(Wf, win_pad, wc1_pad, Wc2)
    u_flat = u2d.reshape(_VPAD)
    vw_flat = vw2d.reshape(2 * HIDDEN)

    conf, fin, loss = _get_sc_main()(ids, cat, y, u_flat, vw_flat)

    confound_pred = conf[:, None]
    final_pred = fin[:, None]
    loss2 = loss.reshape(_NW, 32)
    confound_loss = jnp.sum(loss2[:, :16]) / B
    final_loss = jnp.sum(loss2[:, 16:]) / B
    return (confound_pred, confound_loss, final_pred, final_loss)


# speedup vs baseline: 135.7162x; 1.2688x over previous
import functools

import jax
import jax.numpy as jnp
from jax import lax
from jax.experimental import pallas as pl
from jax.experimental.pallas import tpu as pltpu
from jax.experimental.pallas import tpu_sc as plsc

VOCAB = 1000
HIDDEN = 128
B = 16384
L = 200

_NC = 2
_NS = 16
_NW = _NC * _NS
_R = B // _NW
_NCHUNK = 4
_RC = _R // _NCHUNK
_VPAD = 1024


def _prep_body(wf_ref, win_ref, wc1_ref, wc2_ref, u_ref, vw_ref):
    wf_text = wf_ref[0:1, 0:HIDDEN]
    u = jnp.dot(wf_text, win_ref[...],
                preferred_element_type=jnp.float32,
                precision=lax.Precision.HIGHEST)
    col = lax.broadcasted_iota(jnp.int32, (1, _VPAD), 1)
    u_ref[...] = jnp.where(col == 1, 0.0, u)
    v = jnp.dot(wc2_ref[...], wc1_ref[...],
                preferred_element_type=jnp.float32,
                precision=lax.Precision.HIGHEST)
    colv = lax.broadcasted_iota(jnp.int32, (1, HIDDEN), 1)
    vm = jnp.where(colv == 1, 0.0, v)
    w2 = vm * wf_ref[0:1, HIDDEN:HIDDEN + 1]
    vw_ref[...] = jnp.concatenate([vm, w2], axis=0)


_prep = pl.pallas_call(
    _prep_body,
    out_shape=(
        jax.ShapeDtypeStruct((1, _VPAD), jnp.float32),
        jax.ShapeDtypeStruct((2, HIDDEN), jnp.float32),
    ),
)


def _sc_main_body(ids_hbm, cat_hbm, y_hbm, u_hbm, vw_hbm,
                  conf_hbm, fin_hbm, loss_hbm,
                  ids_a, ids_b, u_v, vw_v, cat_v, y_v, t_v, conf_v, fin_v,
                  loss_v, sem_a, sem_b):
    wid = lax.axis_index("s") * _NC + lax.axis_index("c")
    base = wid * _R

    bufs = [(ids_a, sem_a), (ids_b, sem_b)]
    copies = [None] * _NCHUNK
    copies[0] = pltpu.async_copy(ids_hbm.at[pl.ds(base, _RC)], ids_a, sem_a)
    pltpu.sync_copy(u_hbm, u_v)
    pltpu.sync_copy(vw_hbm, vw_v)
    pltpu.sync_copy(cat_hbm.at[pl.ds(base, _R)], cat_v)
    pltpu.sync_copy(y_hbm.at[pl.ds(base, _R)], y_v)

    lane = lax.iota(jnp.int32, 16)
    tail_mask = lane >= 8
    last_mask = lane == 15

    for chunk in range(_NCHUNK):
        ids_v, _ = bufs[chunk % 2]
        copies[chunk].wait()
        if chunk + 1 < _NCHUNK:
            nbuf, nsem = bufs[(chunk + 1) % 2]
            copies[chunk + 1] = pltpu.async_copy(
                ids_hbm.at[pl.ds(base + (chunk + 1) * _RC, _RC)], nbuf, nsem)
        row0 = chunk * _RC

        def row_body(r, carry, ids_v=ids_v, row0=row0):
            accs = [plsc.load_gather(u_v, [ids_v[r, pl.ds(jv * 16, 16)]])
                    for jv in range(4)]
            for jv in range(4, 12):
                accs[jv % 4] = accs[jv % 4] + plsc.load_gather(
                    u_v, [ids_v[r, pl.ds(jv * 16, 16)]])
            tail = plsc.load_gather(u_v, [ids_v[r, pl.ds(184, 16)]])
            acc = ((accs[0] + accs[1]) + (accs[2] + accs[3])
                   + jnp.where(tail_mask, tail, 0.0))
            csum = plsc.cumsum(acc)
            plsc.store_scatter(t_v, [jnp.full((16,), row0 + r, jnp.int32)],
                               csum, mask=last_mask)
            return carry

        lax.fori_loop(0, _RC, row_body, 0, unroll=2)

    zero16 = jnp.zeros((16,), jnp.float32)

    def ep_body(g, carry):
        lc, lf = carry
        t16 = t_v[pl.ds(g * 16, 16)]
        c16 = cat_v[pl.ds(g * 16, 16)]
        y16 = y_v[pl.ds(g * 16, 16)]
        cpv = plsc.load_gather(vw_v, [c16])
        w2v = plsc.load_gather(vw_v, [c16 + HIDDEN])
        fpv = t16 + w2v
        conf_v[pl.ds(g * 16, 16)] = cpv
        fin_v[pl.ds(g * 16, 16)] = fpv
        dc = cpv - y16
        df = fpv - y16
        return (lc + dc * dc, lf + df * df)

    lc, lf = lax.fori_loop(0, _R // 16, ep_body, (zero16, zero16),
                           unroll=False)
    loss_v[pl.ds(0, 16)] = lc
    loss_v[pl.ds(16, 16)] = lf

    pltpu.sync_copy(conf_v, conf_hbm.at[pl.ds(base, _R)])
    pltpu.sync_copy(fin_v, fin_hbm.at[pl.ds(base, _R)])
    pltpu.sync_copy(loss_v, loss_hbm.at[pl.ds(wid * 32, 32)])


@functools.cache
def _get_sc_main():
    mesh = plsc.VectorSubcoreMesh(core_axis_name="c", subcore_axis_name="s",
                                  num_cores=_NC, num_subcores=_NS)
    return pl.kernel(
        _sc_main_body,
        mesh=mesh,
        compiler_params=pltpu.CompilerParams(needs_layout_passes=False),
        out_type=(
            jax.ShapeDtypeStruct((B,), jnp.float32),
            jax.ShapeDtypeStruct((B,), jnp.float32),
            jax.ShapeDtypeStruct((_NW * 32,), jnp.float32),
        ),
        scratch_types=[
            pltpu.VMEM((_RC, L), jnp.int32),
            pltpu.VMEM((_RC, L), jnp.int32),
            pltpu.VMEM((_VPAD,), jnp.float32),
            pltpu.VMEM((2 * HIDDEN,), jnp.float32),
            pltpu.VMEM((_R,), jnp.int32),
            pltpu.VMEM((_R,), jnp.float32),
            pltpu.VMEM((_R,), jnp.float32),
            pltpu.VMEM((_R,), jnp.float32),
            pltpu.VMEM((_R,), jnp.float32),
            pltpu.VMEM((32,), jnp.float32),
            pltpu.SemaphoreType.DMA,
            pltpu.SemaphoreType.DMA,
        ],
    )


def kernel(input, confound_cat, outcome_y, W_in, Wc1, Wc2, Wf):
    ids = input.astype(jnp.int32)
    cat = confound_cat.astype(jnp.int32)
    y = outcome_y.astype(jnp.float32)

    win_pad = jnp.zeros((HIDDEN, _VPAD), jnp.float32).at[:, :VOCAB].set(W_in)
    wc1_pad = jnp.zeros((HIDDEN, HIDDEN), jnp.float32).at[:, :10].set(Wc1)

    u2d, vw2d = _prep(Wf, win_pad, wc1_pad, Wc2)
    u_flat = u2d.reshape(_VPAD)
    vw_flat = vw2d.reshape(2 * HIDDEN)

    conf, fin, loss = _get_sc_main()(ids, cat, y, u_flat, vw_flat)

    confound_pred = conf[:, None]
    final_pred = fin[:, None]
    loss2 = loss.reshape(_NW, 32)
    confound_loss = jnp.sum(loss2[:, :16]) / B
    final_loss = jnp.sum(loss2[:, 16:]) / B
    return (confound_pred, confound_loss, final_pred, final_loss)
